# baseline (speedup 1.0000x reference)
import jax
import jax.numpy as jnp
from jax import lax
from jax.experimental import pallas as pl
from jax.experimental.pallas import tpu as pltpu
from jax.experimental.pallas import tpu_sc as plsc

MAX_LEN = 4096
D_MODEL = 1024
BATCH = 4

NUM_CORES = 2
NUM_SUBCORES = 16
NUM_WORKERS = NUM_CORES * NUM_SUBCORES
ROWS_PER_WORKER = MAX_LEN // NUM_WORKERS
CHUNK = 32
NUM_CHUNKS = ROWS_PER_WORKER // CHUNK


def _sc_broadcast(table_hbm, out_hbm, buf0, buf1, buf2, gsem, ssem0, ssem1, ssem2):
    wid = lax.axis_index("s") * NUM_CORES + lax.axis_index("c")
    base = wid * ROWS_PER_WORKER
    bufs = (buf0, buf1, buf2)
    ssems = (ssem0, ssem1, ssem2)

    def gather(c):
        row = base + c * CHUNK
        return pltpu.async_copy(table_hbm.at[pl.ds(row, CHUNK)], bufs[c % 3], gsem)

    gathers = {0: gather(0)}
    scatters = {}
    for c in range(NUM_CHUNKS):
        row = base + c * CHUNK
        gathers[c].wait()
        scatters[c] = [
            pltpu.async_copy(bufs[c % 3], out_hbm.at[b, pl.ds(row, CHUNK)], ssems[c % 3])
            for b in range(BATCH)
        ]
        if c + 1 < NUM_CHUNKS:
            if c - 2 >= 0:
                for cp in scatters[c - 2]:
                    cp.wait()
            gathers[c + 1] = gather(c + 1)
    for c in range(max(0, NUM_CHUNKS - 2), NUM_CHUNKS):
        for cp in scatters[c]:
            cp.wait()


def _sc_broadcast_direct(table_hbm, out_hbm, sem):
    wid = lax.axis_index("s") * NUM_CORES + lax.axis_index("c")
    base = wid * ROWS_PER_WORKER
    sl = pl.ds(base, ROWS_PER_WORKER)
    copies = [
        pltpu.async_copy(table_hbm.at[sl], out_hbm.at[b, sl], sem)
        for b in range(BATCH)
    ]
    for cp in copies:
        cp.wait()


TC_CHUNK = 512
TC_NCHUNK = MAX_LEN // TC_CHUNK


def _tc_dma_body(table_hbm, out_hbm, buf, gsem, ssem):
    for c in range(TC_NCHUNK):
        row = c * TC_CHUNK
        g = pltpu.make_async_copy(table_hbm.at[pl.ds(row, TC_CHUNK)], buf, gsem)
        g.start()
        g.wait()
        scatters = []
        for b in range(BATCH):
            cp = pltpu.make_async_copy(
                buf, out_hbm.at[pl.ds(b * MAX_LEN + row, TC_CHUNK)], ssem)
            cp.start()
            scatters.append(cp)
        for cp in scatters:
            cp.wait()


def _sc_batch3(table_hbm, out_hbm, buf0, buf1, gsem, ssem0, ssem1):
    wid = lax.axis_index("s") * NUM_CORES + lax.axis_index("c")
    base = wid * ROWS_PER_WORKER
    bufs = (buf0, buf1)
    ssems = (ssem0, ssem1)
    n = ROWS_PER_WORKER // CHUNK

    def gather(c):
        return pltpu.async_copy(
            table_hbm.at[pl.ds(base + c * CHUNK, CHUNK)], bufs[c % 2], gsem)

    gathers = {0: gather(0)}
    scatters = {}
    for c in range(n):
        gathers[c].wait()
        scatters[c] = pltpu.async_copy(
            bufs[c % 2], out_hbm.at[pl.ds(base + c * CHUNK, CHUNK)], ssems[c % 2])
        if c + 1 < n:
            if c - 1 >= 0:
                scatters[c - 1].wait()
            gathers[c + 1] = gather(c + 1)
    for c in range(max(0, n - 2), n):
        scatters[c].wait()


def _tc_batches012(table_hbm, out_hbm, buf, gsem, ssem):
    for c in range(TC_NCHUNK):
        row = c * TC_CHUNK
        g = pltpu.make_async_copy(table_hbm.at[pl.ds(row, TC_CHUNK)], buf, gsem)
        g.start()
        g.wait()
        scatters = []
        for b in range(BATCH - 1):
            cp = pltpu.make_async_copy(
                buf, out_hbm.at[pl.ds(b * MAX_LEN + row, TC_CHUNK)], ssem)
            cp.start()
            scatters.append(cp)
        for cp in scatters:
            cp.wait()


def kernel(x, emb_weight):
    del x
    tc_part = pl.pallas_call(
        _tc_batches012,
        in_specs=[pl.BlockSpec(memory_space=pl.ANY)],
        out_specs=pl.BlockSpec(memory_space=pl.ANY),
        out_shape=jax.ShapeDtypeStruct(((BATCH - 1) * MAX_LEN, D_MODEL), jnp.float32),
        scratch_shapes=[
            pltpu.VMEM((TC_CHUNK, D_MODEL), jnp.float32),
            pltpu.SemaphoreType.DMA,
            pltpu.SemaphoreType.DMA,
        ],
    )(emb_weight)
    sc_part = pl.kernel(
        _sc_batch3,
        out_type=jax.ShapeDtypeStruct((MAX_LEN, D_MODEL), jnp.float32),
        mesh=plsc.VectorSubcoreMesh(core_axis_name="c", subcore_axis_name="s"),
        scratch_types=[
            pltpu.VMEM((CHUNK, D_MODEL), jnp.float32),
            pltpu.VMEM((CHUNK, D_MODEL), jnp.float32),
            pltpu.SemaphoreType.DMA,
            pltpu.SemaphoreType.DMA,
            pltpu.SemaphoreType.DMA,
        ],
    )(emb_weight)
    return jnp.concatenate(
        [tc_part.reshape(BATCH - 1, MAX_LEN, D_MODEL), sc_part[None]], axis=0)


# speedup vs baseline: 2.1916x; 2.1916x over previous
import jax
import jax.numpy as jnp
from jax import lax
from jax.experimental import pallas as pl
from jax.experimental.pallas import tpu as pltpu
from jax.experimental.pallas import tpu_sc as plsc

NUM_CORES = 2
NUM_SUBCORES = 16
NUM_WORKERS = NUM_CORES * NUM_SUBCORES
CHUNK = 64


def _make_sc_broadcast(batch, rows_per_worker):
    num_chunks = rows_per_worker // CHUNK

    def body(table_hbm, out_hbm, buf, sem):
        wid = lax.axis_index("s") * NUM_CORES + lax.axis_index("c")
        base = wid * rows_per_worker
        for c in range(num_chunks):
            row = base + c * CHUNK
            pltpu.sync_copy(table_hbm.at[pl.ds(row, CHUNK)], buf)
            copies = [
                pltpu.async_copy(buf, out_hbm.at[b, pl.ds(row, CHUNK)], sem)
                for b in range(batch)
            ]
            for cp in copies:
                cp.wait()

    return body


def kernel(x, emb_weight):
    batch = x.shape[0]
    max_len, d_model = emb_weight.shape
    assert max_len % (NUM_WORKERS * CHUNK) == 0
    f = pl.kernel(
        _make_sc_broadcast(batch, max_len // NUM_WORKERS),
        out_type=jax.ShapeDtypeStruct((batch, max_len, d_model), jnp.float32),
        mesh=plsc.VectorSubcoreMesh(core_axis_name="c", subcore_axis_name="s"),
        scratch_types=[
            pltpu.VMEM((CHUNK, d_model), jnp.float32),
            pltpu.SemaphoreType.DMA,
        ],
    )
    return f(emb_weight)
